# baseline (device time: 1807091 ns/iter reference)
import jax
import jax.numpy as jnp
from jax import lax
from jax.experimental import pallas as pl
from jax.experimental.pallas import tpu as pltpu

N_ROWS = 4096
N_COLS = 4096
Q_ROWS = N_ROWS // 4
NC = 8
CH = Q_ROWS // NC


def kernel(partial, resid, gamma):
    g = gamma.reshape(1, N_COLS)

    def body(p_ref, resid_ref, g_ref, out_ref, stage_ref,
             p_vmem, s_vmem, resid_vmem, o_vmem,
             p1_send, p1_recv, a_send, a_recv,
             b_own_send, b_own_recv, b_fwd_send, b_fwd_recv,
             copy_sems):
        my_x = lax.axis_index("x")
        my_y = lax.axis_index("y")
        y_nbr = (my_x, 1 - my_y)
        x_nbr = (1 - my_x, my_y)

        q_me = 2 * my_x + my_y
        q_yn = 2 * my_x + (1 - my_y)
        q_xn = 2 * (1 - my_x) + my_y
        q_dg = 2 * (1 - my_x) + (1 - my_y)
        r_me = q_me * Q_ROWS
        r_yn = q_yn * Q_ROWS

        def rdma(src, dst, ssem, rsem, dev):
            return pltpu.make_async_remote_copy(
                src_ref=src, dst_ref=dst, send_sem=ssem, recv_sem=rsem,
                device_id=dev, device_id_type=pl.DeviceIdType.MESH)

        barrier = pltpu.get_barrier_semaphore()
        for nbr in (y_nbr, x_nbr):
            pl.semaphore_signal(barrier, inc=1, device_id=nbr,
                                device_id_type=pl.DeviceIdType.MESH)
        pl.semaphore_wait(barrier, 2)

        for c in range(NC):
            rows = pl.ds(r_yn + c * CH, CH)
            rdma(p_ref.at[0, rows], stage_ref.at[rows],
                 p1_send.at[c], p1_recv.at[c], y_nbr).start()

        for c in range(NC):
            rows = pl.ds(r_me + c * CH, CH)
            rdma(p_ref.at[0, rows], stage_ref.at[rows],
                 p1_send.at[c], p1_recv.at[c], y_nbr).wait_recv()

            cp_p = pltpu.make_async_copy(p_ref.at[0, rows], p_vmem, copy_sems.at[0])
            cp_s = pltpu.make_async_copy(stage_ref.at[rows], s_vmem, copy_sems.at[1])
            cp_r = pltpu.make_async_copy(resid_ref.at[rows], resid_vmem, copy_sems.at[2])
            cp_p.start(); cp_s.start(); cp_r.start()
            cp_p.wait(); cp_s.wait(); cp_r.wait()

            yv = p_vmem[...] + s_vmem[...] + resid_vmem[...]
            rms = jnp.sqrt(jnp.mean(yv * yv, axis=-1, keepdims=True) + 1e-6)
            o_vmem[...] = yv / rms * g_ref[...]

            cp_o = pltpu.make_async_copy(o_vmem, out_ref.at[rows], copy_sems.at[3])
            cp_o.start(); cp_o.wait()

            rdma(out_ref.at[rows], stage_ref.at[rows],
                 a_send.at[c], a_recv.at[c], y_nbr).start()
            rdma(out_ref.at[rows], stage_ref.at[rows],
                 b_own_send.at[c], b_own_recv.at[c], x_nbr).start()

        for c in range(NC):
            rows = pl.ds(r_yn + c * CH, CH)
            rdma(stage_ref.at[rows], stage_ref.at[rows],
                 a_send.at[c], a_recv.at[c], y_nbr).wait_recv()
            rdma(stage_ref.at[rows], stage_ref.at[rows],
                 b_fwd_send.at[c], b_fwd_recv.at[c], x_nbr).start()
            cp = pltpu.make_async_copy(stage_ref.at[rows], out_ref.at[rows],
                                       copy_sems.at[4])
            cp.start(); cp.wait()

        for c in range(NC):
            rows = pl.ds(q_xn * Q_ROWS + c * CH, CH)
            rdma(stage_ref.at[rows], stage_ref.at[rows],
                 b_own_send.at[c], b_own_recv.at[c], x_nbr).wait_recv()
            cp = pltpu.make_async_copy(stage_ref.at[rows], out_ref.at[rows],
                                       copy_sems.at[5])
            cp.start(); cp.wait()
            rows = pl.ds(q_dg * Q_ROWS + c * CH, CH)
            rdma(stage_ref.at[rows], stage_ref.at[rows],
                 b_fwd_recv.at[c], b_fwd_recv.at[c], x_nbr).wait_recv()
            cp = pltpu.make_async_copy(stage_ref.at[rows], out_ref.at[rows],
                                       copy_sems.at[6])
            cp.start(); cp.wait()

        for c in range(NC):
            rows = pl.ds(r_yn + c * CH, CH)
            rdma(p_ref.at[0, rows], stage_ref.at[rows],
                 p1_send.at[c], p1_recv.at[c], y_nbr).wait_send()
            rdma(stage_ref.at[rows], stage_ref.at[rows],
                 b_fwd_send.at[c], b_fwd_recv.at[c], x_nbr).wait_send()
            rows = pl.ds(r_me + c * CH, CH)
            rdma(out_ref.at[rows], stage_ref.at[rows],
                 a_send.at[c], a_recv.at[c], y_nbr).wait_send()
            rdma(out_ref.at[rows], stage_ref.at[rows],
                 b_own_send.at[c], b_own_recv.at[c], x_nbr).wait_send()

    sem_arr = pltpu.SemaphoreType.DMA((NC,))
    out, _stage = pl.pallas_call(
        body,
        out_shape=[
            jax.ShapeDtypeStruct((N_ROWS, N_COLS), jnp.float32),
            jax.ShapeDtypeStruct((N_ROWS, N_COLS), jnp.float32),
        ],
        in_specs=[
            pl.BlockSpec(memory_space=pl.MemorySpace.ANY),
            pl.BlockSpec(memory_space=pl.MemorySpace.ANY),
            pl.BlockSpec(memory_space=pltpu.MemorySpace.VMEM),
        ],
        out_specs=[
            pl.BlockSpec(memory_space=pl.MemorySpace.ANY),
            pl.BlockSpec(memory_space=pl.MemorySpace.ANY),
        ],
        scratch_shapes=[
            pltpu.VMEM((CH, N_COLS), jnp.float32),
            pltpu.VMEM((CH, N_COLS), jnp.float32),
            pltpu.VMEM((CH, N_COLS), jnp.float32),
            pltpu.VMEM((CH, N_COLS), jnp.float32),
            sem_arr, sem_arr, sem_arr, sem_arr,
            sem_arr, sem_arr, sem_arr, sem_arr,
            pltpu.SemaphoreType.DMA((7,)),
        ],
        compiler_params=pltpu.CompilerParams(
            collective_id=0, vmem_limit_bytes=96 * 1024 * 1024),
    )(partial, resid, g)
    return out


# device time: 434477 ns/iter; 4.1592x vs baseline; 4.1592x over previous
import jax
import jax.numpy as jnp
from jax import lax
from jax.experimental import pallas as pl
from jax.experimental.pallas import tpu as pltpu

N_ROWS = 4096
N_COLS = 4096
Q_ROWS = N_ROWS // 4
NC = 8
CH = Q_ROWS // NC
LAST = NC - 1


def kernel(partial, resid, gamma):
    g = gamma.reshape(1, N_COLS)

    def body(p_ref, resid_ref, g_ref, out_ref,
             p_vmem, s_vmem, resid_vmem, o_vmem,
             p1_send, p1_recv, a_send, a_recv,
             b_own_send, b_own_recv, b_fwd_send, b_fwd_recv,
             c_send, c_recv, copy_sems):
        my_x = lax.axis_index("x")
        my_y = lax.axis_index("y")
        y_nbr = (my_x, 1 - my_y)
        x_nbr = (1 - my_x, my_y)

        q_me = 2 * my_x + my_y
        q_yn = 2 * my_x + (1 - my_y)
        q_xn = 2 * (1 - my_x) + my_y
        q_dg = 2 * (1 - my_x) + (1 - my_y)
        r_me = q_me * Q_ROWS
        r_yn = q_yn * Q_ROWS

        def rdma(src, dst, ssem, rsem, dev):
            return pltpu.make_async_remote_copy(
                src_ref=src, dst_ref=dst, send_sem=ssem, recv_sem=rsem,
                device_id=dev, device_id_type=pl.DeviceIdType.MESH)

        barrier = pltpu.get_barrier_semaphore()
        for nbr in (y_nbr, x_nbr):
            pl.semaphore_signal(barrier, inc=1, device_id=nbr,
                                device_id_type=pl.DeviceIdType.MESH)
        pl.semaphore_wait(barrier, 2)

        for c in range(NC):
            rows = pl.ds(r_yn + c * CH, CH)
            rdma(p_ref.at[0, rows], out_ref.at[rows],
                 p1_send.at[c], p1_recv.at[c], y_nbr).start()

        for c in range(NC):
            rows = pl.ds(r_me + c * CH, CH)
            cp_p = pltpu.make_async_copy(p_ref.at[0, rows], p_vmem, copy_sems.at[0])
            cp_r = pltpu.make_async_copy(resid_ref.at[rows], resid_vmem, copy_sems.at[2])
            cp_p.start(); cp_r.start()
            rdma(p_ref.at[0, rows], out_ref.at[rows],
                 p1_send.at[c], p1_recv.at[c], y_nbr).wait_recv()
            cp_s = pltpu.make_async_copy(out_ref.at[rows], s_vmem, copy_sems.at[1])
            cp_s.start()
            cp_p.wait(); cp_r.wait(); cp_s.wait()

            yv = p_vmem[...] + s_vmem[...] + resid_vmem[...]
            rms = jnp.sqrt(jnp.mean(yv * yv, axis=-1, keepdims=True) + 1e-6)
            o_vmem[...] = yv / rms * g_ref[...]

            cp_o = pltpu.make_async_copy(o_vmem, out_ref.at[rows], copy_sems.at[3])
            cp_o.start(); cp_o.wait()

            rdma(out_ref.at[rows], out_ref.at[rows],
                 a_send.at[c], a_recv.at[c], y_nbr).start()
            rdma(out_ref.at[rows], out_ref.at[rows],
                 b_own_send.at[c], b_own_recv.at[c], x_nbr).start()

        for c in range(NC):
            rows = pl.ds(r_yn + c * CH, CH)
            rdma(out_ref.at[rows], out_ref.at[rows],
                 a_send.at[c], a_recv.at[c], y_nbr).wait_recv()
            if c != LAST:
                rdma(out_ref.at[rows], out_ref.at[rows],
                     b_fwd_send.at[c], b_fwd_recv.at[c], x_nbr).start()

        for c in range(NC):
            rows = pl.ds(q_xn * Q_ROWS + c * CH, CH)
            rdma(out_ref.at[rows], out_ref.at[rows],
                 b_own_send.at[c], b_own_recv.at[c], x_nbr).wait_recv()
        rows_c = pl.ds(q_xn * Q_ROWS + LAST * CH, CH)
        rdma(out_ref.at[rows_c], out_ref.at[rows_c],
             c_send, c_recv, y_nbr).start()

        for c in range(NC - 1):
            rows = pl.ds(q_dg * Q_ROWS + c * CH, CH)
            rdma(out_ref.at[rows], out_ref.at[rows],
                 b_fwd_send.at[c], b_fwd_recv.at[c], x_nbr).wait_recv()
        rows_d = pl.ds(q_dg * Q_ROWS + LAST * CH, CH)
        rdma(out_ref.at[rows_d], out_ref.at[rows_d],
             c_send, c_recv, y_nbr).wait_recv()

        rdma(out_ref.at[rows_c], out_ref.at[rows_c],
             c_send, c_recv, y_nbr).wait_send()
        for c in range(NC):
            rows = pl.ds(r_yn + c * CH, CH)
            rdma(p_ref.at[0, rows], out_ref.at[rows],
                 p1_send.at[c], p1_recv.at[c], y_nbr).wait_send()
            if c != LAST:
                rdma(out_ref.at[rows], out_ref.at[rows],
                     b_fwd_send.at[c], b_fwd_recv.at[c], x_nbr).wait_send()
            rows = pl.ds(r_me + c * CH, CH)
            rdma(out_ref.at[rows], out_ref.at[rows],
                 a_send.at[c], a_recv.at[c], y_nbr).wait_send()
            rdma(out_ref.at[rows], out_ref.at[rows],
                 b_own_send.at[c], b_own_recv.at[c], x_nbr).wait_send()

    sem_arr = pltpu.SemaphoreType.DMA((NC,))
    return pl.pallas_call(
        body,
        out_shape=jax.ShapeDtypeStruct((N_ROWS, N_COLS), jnp.float32),
        in_specs=[
            pl.BlockSpec(memory_space=pl.MemorySpace.ANY),
            pl.BlockSpec(memory_space=pl.MemorySpace.ANY),
            pl.BlockSpec(memory_space=pltpu.MemorySpace.VMEM),
        ],
        out_specs=pl.BlockSpec(memory_space=pl.MemorySpace.ANY),
        scratch_shapes=[
            pltpu.VMEM((CH, N_COLS), jnp.float32),
            pltpu.VMEM((CH, N_COLS), jnp.float32),
            pltpu.VMEM((CH, N_COLS), jnp.float32),
            pltpu.VMEM((CH, N_COLS), jnp.float32),
            sem_arr, sem_arr, sem_arr, sem_arr,
            sem_arr, sem_arr, sem_arr, sem_arr,
            pltpu.SemaphoreType.DMA,
            pltpu.SemaphoreType.DMA,
            pltpu.SemaphoreType.DMA((4,)),
        ],
        compiler_params=pltpu.CompilerParams(
            collective_id=0, vmem_limit_bytes=96 * 1024 * 1024),
    )(partial, resid, g)
